# baseline (speedup 1.0000x reference)
import functools
import math

import jax
import jax.numpy as jnp
from jax import lax
from jax.experimental import pallas as pl
from jax.experimental.pallas import tpu as pltpu
from jax.experimental.pallas import tpu_sc as plsc

VOCAB = 100000
D_TOK = 56
D_TYPE = 8
D_MODEL = 64
D_PAD = 128
ROW_BLOCK = 10000
N_BLOCKS = VOCAB // ROW_BLOCK


def _table_body(tok_ref, type_ref, gamma_ref, beta_ref, out_ref):
    i = pl.program_id(0)
    t = (i >= 5).astype(jnp.int32) + (i >= 6).astype(jnp.int32) + (i >= 8).astype(jnp.int32)
    typ = type_ref[...]
    row = jnp.zeros((1, D_TYPE), jnp.float32)
    for k in range(4):
        row = jnp.where(t == k, typ[k : k + 1, :], row)
    combined = jnp.concatenate(
        [tok_ref[...], jnp.broadcast_to(row, (ROW_BLOCK, D_TYPE))], axis=-1
    )
    mean = jnp.mean(combined, axis=-1, keepdims=True)
    var = jnp.mean((combined - mean) ** 2, axis=-1, keepdims=True)
    rstd = lax.rsqrt(var + 1e-5)
    normed = ((combined - mean) * rstd * gamma_ref[...] + beta_ref[...]) * math.sqrt(
        float(D_MODEL)
    )
    out_ref[...] = jnp.concatenate(
        [normed, jnp.zeros((ROW_BLOCK, D_PAD - D_MODEL), jnp.float32)], axis=-1
    )


def _build_table(token_table, type_table, ln_gamma, ln_beta):
    return pl.pallas_call(
        _table_body,
        grid=(N_BLOCKS,),
        in_specs=[
            pl.BlockSpec((ROW_BLOCK, D_TOK), lambda i: (i, 0)),
            pl.BlockSpec((4, D_TYPE), lambda i: (0, 0)),
            pl.BlockSpec((1, D_MODEL), lambda i: (0, 0)),
            pl.BlockSpec((1, D_MODEL), lambda i: (0, 0)),
        ],
        out_specs=pl.BlockSpec((ROW_BLOCK, D_PAD), lambda i: (i, 0)),
        out_shape=jax.ShapeDtypeStruct((VOCAB, D_PAD), jnp.float32),
    )(token_table, type_table, ln_gamma.reshape(1, D_MODEL), ln_beta.reshape(1, D_MODEL))


_NC = 2
_NS = 16
_NW = _NC * _NS
_L = 16
_BSTRIPE = 128
_SC = 2


def _sc_gather_t(table, xT, B, S):
    n_chunks = S // _SC

    mesh = plsc.VectorSubcoreMesh(core_axis_name="c", subcore_axis_name="s")

    @functools.partial(
        pl.kernel,
        mesh=mesh,
        out_type=jax.ShapeDtypeStruct((S, D_MODEL, B), jnp.float32),
        compiler_params=pltpu.CompilerParams(needs_layout_passes=False),
        scratch_types=[
            pltpu.VMEM((S, _BSTRIPE), jnp.int32),
            pltpu.VMEM((2, _SC, _BSTRIPE, D_PAD), jnp.float32),
            pltpu.VMEM((2, _SC, D_MODEL, _BSTRIPE), jnp.float32),
            pltpu.SemaphoreType.DMA,
            pltpu.SemaphoreType.DMA,
        ],
    )
    def k(table_hbm, xT_hbm, out_hbm, idx_v, rows_v, tbuf_v, gsem, wsem):
        wid = lax.axis_index("s") * _NC + lax.axis_index("c")
        b0 = wid * _BSTRIPE
        pltpu.sync_copy(xT_hbm.at[:, pl.ds(b0, _BSTRIPE)], idx_v)

        iota = lax.iota(jnp.int32, _L)
        xor_idx = [iota ^ m for m in (1, 2, 4, 8)]
        xor_msk = [(iota & m) != 0 for m in (1, 2, 4, 8)]

        def fire(ch, buf):
            for i in range(_SC):
                pltpu.async_copy(
                    table_hbm.at[idx_v.at[_SC * ch + i]],
                    rows_v.at[buf, i],
                    gsem,
                )

        fire(0, 0)

        def body(t, carry):
            for buf in range(2):
                ch = 2 * t + buf
                for i in range(_SC):
                    pltpu.make_async_copy(
                        table_hbm.at[idx_v.at[0]], rows_v.at[buf, i], gsem
                    ).wait()

                @pl.when(ch + 1 < n_chunks)
                def _():
                    fire(ch + 1, 1 - buf)

                @pl.when(ch >= 2)
                def _():
                    pltpu.make_async_copy(
                        out_hbm.at[pl.ds(0, _SC), :, pl.ds(b0, _BSTRIPE)],
                        tbuf_v.at[buf],
                        wsem,
                    ).wait()

                n_tiles = (_BSTRIPE // _L) * _SC * (D_MODEL // _L)

                @plsc.parallel_loop(0, n_tiles, unroll=2)
                def tile_body(t):
                    kk = t >> 3
                    i = (t >> 2) & 1
                    c = t & 3
                    r0 = 16 * kk
                    c0 = 16 * c
                    x = [
                        rows_v[buf, i, r0 + r, pl.ds(c0, _L)] for r in range(_L)
                    ]
                    for si, m in enumerate((1, 2, 4, 8)):
                        idxm, mskm = xor_idx[si], xor_msk[si]
                        for r in range(_L):
                            if r & m == 0:
                                p = r | m
                                a, b = x[r], x[p]
                                u = jnp.where(mskm, a, b)
                                ush = u.at[idxm].get(mode="promise_in_bounds")
                                x[r] = jnp.where(mskm, ush, a)
                                x[p] = jnp.where(mskm, b, ush)
                    for q in range(_L):
                        tbuf_v[buf, i, c0 + q, pl.ds(r0, _L)] = x[q]

                pltpu.async_copy(
                    tbuf_v.at[buf],
                    out_hbm.at[pl.ds(_SC * ch, _SC), :, pl.ds(b0, _BSTRIPE)],
                    wsem,
                )
            return carry

        lax.fori_loop(0, n_chunks // 2, body, 0)

        for buf in range(2):
            pltpu.make_async_copy(
                out_hbm.at[pl.ds(0, _SC), :, pl.ds(b0, _BSTRIPE)],
                tbuf_v.at[buf],
                wsem,
            ).wait()

    return k(table, xT)


def kernel(x, token_table, type_table, ln_gamma, ln_beta):
    b, s = x.shape
    table = _build_table(token_table, type_table, ln_gamma, ln_beta)
    xT = jnp.transpose(x.astype(jnp.int32))
    out3 = _sc_gather_t(table, xT, b, s)
    return jnp.transpose(out3, (2, 0, 1))


# speedup vs baseline: 1.1350x; 1.1350x over previous
import functools
import math

import jax
import jax.numpy as jnp
from jax import lax
from jax.experimental import pallas as pl
from jax.experimental.pallas import tpu as pltpu
from jax.experimental.pallas import tpu_sc as plsc

VOCAB = 100000
D_TOK = 56
D_TYPE = 8
D_MODEL = 64
D_PAD = 128
ROW_BLOCK = 10000
N_BLOCKS = VOCAB // ROW_BLOCK


def _table_body(tok_ref, type_ref, gamma_ref, beta_ref, out_ref):
    i = pl.program_id(0)
    t = (i >= 5).astype(jnp.int32) + (i >= 6).astype(jnp.int32) + (i >= 8).astype(jnp.int32)
    typ = type_ref[...]
    row = jnp.zeros((1, D_TYPE), jnp.float32)
    for k in range(4):
        row = jnp.where(t == k, typ[k : k + 1, :], row)
    combined = jnp.concatenate(
        [tok_ref[...], jnp.broadcast_to(row, (ROW_BLOCK, D_TYPE))], axis=-1
    )
    mean = jnp.mean(combined, axis=-1, keepdims=True)
    var = jnp.mean((combined - mean) ** 2, axis=-1, keepdims=True)
    rstd = lax.rsqrt(var + 1e-5)
    out_ref[...] = ((combined - mean) * rstd * gamma_ref[...] + beta_ref[...]) * math.sqrt(
        float(D_MODEL)
    )


def _build_table(token_table, type_table, ln_gamma, ln_beta):
    return pl.pallas_call(
        _table_body,
        grid=(N_BLOCKS,),
        in_specs=[
            pl.BlockSpec((ROW_BLOCK, D_TOK), lambda i: (i, 0)),
            pl.BlockSpec((4, D_TYPE), lambda i: (0, 0)),
            pl.BlockSpec((1, D_MODEL), lambda i: (0, 0)),
            pl.BlockSpec((1, D_MODEL), lambda i: (0, 0)),
        ],
        out_specs=pl.BlockSpec((ROW_BLOCK, D_MODEL), lambda i: (i, 0)),
        out_shape=jax.ShapeDtypeStruct((VOCAB, D_MODEL), jnp.float32),
    )(token_table, type_table, ln_gamma.reshape(1, D_MODEL), ln_beta.reshape(1, D_MODEL))


_NC = 2
_NS = 16
_NW = _NC * _NS
_L = 16
_BSTRIPE = 128
_SC = 2


def _sc_gather_t(table, xT, B, S):
    n_chunks = S // _SC

    mesh = plsc.VectorSubcoreMesh(core_axis_name="c", subcore_axis_name="s")

    @functools.partial(
        pl.kernel,
        mesh=mesh,
        out_type=jax.ShapeDtypeStruct(
            (S, D_MODEL // 8, B // _BSTRIPE, 8, _BSTRIPE), jnp.float32
        ),
        compiler_params=pltpu.CompilerParams(
            use_tc_tiling_on_sc=False, needs_layout_passes=False
        ),
        scratch_types=[
            pltpu.VMEM((S, _BSTRIPE), jnp.int32),
            pltpu.VMEM((2, _SC, _BSTRIPE, D_MODEL), jnp.float32),
            pltpu.VMEM((2, _SC, D_MODEL // 8, 8, _BSTRIPE), jnp.float32),
            pltpu.SemaphoreType.DMA,
            pltpu.SemaphoreType.DMA,
        ],
    )
    def k(table_hbm, xT_hbm, out_hbm, idx_v, rows_v, tbuf_v, gsem, wsem):
        wid = lax.axis_index("s") * _NC + lax.axis_index("c")
        b0 = wid * _BSTRIPE
        pltpu.sync_copy(xT_hbm.at[:, pl.ds(b0, _BSTRIPE)], idx_v)

        iota = lax.iota(jnp.int32, _L)
        xor_idx = [iota ^ m for m in (1, 2, 4, 8)]
        xor_msk = [(iota & m) != 0 for m in (1, 2, 4, 8)]

        def fire(ch, buf):
            for i in range(_SC):
                pltpu.async_copy(
                    table_hbm.at[idx_v.at[_SC * ch + i]],
                    rows_v.at[buf, i],
                    gsem,
                )

        fire(0, 0)

        def body(t, carry):
            for buf in range(2):
                ch = 2 * t + buf
                for i in range(_SC):
                    pltpu.make_async_copy(
                        table_hbm.at[idx_v.at[0]], rows_v.at[buf, i], gsem
                    ).wait()

                @pl.when(ch + 1 < n_chunks)
                def _():
                    fire(ch + 1, 1 - buf)

                @pl.when(ch >= 2)
                def _():
                    pltpu.make_async_copy(
                        out_hbm.at[pl.ds(0, _SC), :, 0],
                        tbuf_v.at[buf],
                        wsem,
                    ).wait()

                n_tiles = (_BSTRIPE // _L) * _SC * (D_MODEL // _L)

                @plsc.parallel_loop(0, n_tiles, unroll=2)
                def tile_body(t):
                    kk = t >> 3
                    i = (t >> 2) & 1
                    c = t & 3
                    r0 = 16 * kk
                    c0 = 16 * c
                    x = [
                        rows_v[buf, i, r0 + r, pl.ds(c0, _L)] for r in range(_L)
                    ]
                    for si, m in enumerate((1, 2, 4, 8)):
                        idxm, mskm = xor_idx[si], xor_msk[si]
                        for r in range(_L):
                            if r & m == 0:
                                p = r | m
                                a, b = x[r], x[p]
                                u = jnp.where(mskm, a, b)
                                ush = u.at[idxm].get(mode="promise_in_bounds")
                                x[r] = jnp.where(mskm, ush, a)
                                x[p] = jnp.where(mskm, b, ush)
                    for q in range(_L):
                        d = c0 + q
                        tbuf_v[buf, i, d >> 3, d & 7, pl.ds(r0, _L)] = x[q]

                pltpu.async_copy(
                    tbuf_v.at[buf],
                    out_hbm.at[pl.ds(_SC * ch, _SC), :, wid],
                    wsem,
                )
            return carry

        lax.fori_loop(0, n_chunks // 2, body, 0)

        for buf in range(2):
            pltpu.make_async_copy(
                out_hbm.at[pl.ds(0, _SC), :, 0],
                tbuf_v.at[buf],
                wsem,
            ).wait()

    return k(table, xT)


def kernel(x, token_table, type_table, ln_gamma, ln_beta):
    b, s = x.shape
    table = _build_table(token_table, type_table, ln_gamma, ln_beta)
    xT = jnp.transpose(x.astype(jnp.int32))
    out5 = _sc_gather_t(table, xT, b, s)
    return jnp.transpose(out5, (2, 4, 0, 1, 3)).reshape(b, s, D_MODEL)
